# Initial kernel scaffold; baseline (speedup 1.0000x reference)
import jax
import jax.numpy as jnp
from jax.experimental import pallas as pl


def kernel(xt, edge_index, log_w, B, W0, b0, W1, b1, W2, b2, W3, b3):
    raise NotImplementedError("write your pallas kernel here")


# speedup vs baseline: 139.4081x; 139.4081x over previous
import functools

import jax
import jax.numpy as jnp
from jax import lax
from jax.experimental import pallas as pl
from jax.experimental.pallas import tpu as pltpu
from jax.experimental.pallas import tpu_sc as plsc

N = 100000
E = 6400000
NF = 64
H = 128
K = 3

N_PAD = 102400
ROWS = 2048
SLICE = N_PAD // 16
CSLICE = SLICE // 4
GSLICE = N_PAD // 32
EPW = E // 32
EC = 2000
NCH = EPW // EC

_f32 = jnp.float32


def _mlp_body(x_ref, b_ref, w0_ref, b0_ref, w1_ref, b1_ref, w2_ref, b2_ref,
              w3_ref, b3_ref, o_ref):
    x = x_ref[...]
    proj = jnp.dot(x, b_ref[...], preferred_element_type=_f32)
    feat = jnp.concatenate([jnp.sin(proj), jnp.cos(proj)], axis=-1)
    h = jnp.tanh(jnp.dot(feat, w0_ref[...], preferred_element_type=_f32)
                 + b0_ref[...])
    h = jnp.tanh(jnp.dot(h, w1_ref[...], preferred_element_type=_f32)
                 + b1_ref[...])
    h = jnp.tanh(jnp.dot(h, w2_ref[...], preferred_element_type=_f32)
                 + b2_ref[...])
    o_ref[...] = jnp.dot(h, w3_ref[...], preferred_element_type=_f32) + b3_ref[...]


def _mlp(xt_pad, B, W0, b0, W1, b1, W2, b2, W3, b3):
    grid = (N_PAD // ROWS,)
    full = lambda r, c: pl.BlockSpec((r, c), lambda i: (0, 0))
    return pl.pallas_call(
        _mlp_body,
        grid=grid,
        in_specs=[
            pl.BlockSpec((ROWS, 2), lambda i: (i, 0)),
            full(2, NF), full(2 * NF, H), full(1, H), full(H, H), full(1, H),
            full(H, H), full(1, H), full(H, 1), full(1, 1),
        ],
        out_specs=pl.BlockSpec((ROWS, 1), lambda i: (i, 0)),
        out_shape=jax.ShapeDtypeStruct((N_PAD, 1), _f32),
    )(xt_pad, B, W0, b0.reshape(1, H), W1, b1.reshape(1, H),
      W2, b2.reshape(1, H), W3, b3.reshape(1, 1))


def _hop_body(u_hbm, aa_hbm, ab_hbm, ei_hbm, lw_hbm,
              ucomb_hbm, a0_hbm, a1_hbm,
              u_tab, b_src, b_dst, b_lw, b_msg, b_tmp, u_sh, acc_sh):
    c = lax.axis_index("c")
    s = lax.axis_index("s")

    for j in range(SLICE // CSLICE):
        off = s * SLICE + j * CSLICE
        csl = pl.ds(0, CSLICE)
        pltpu.sync_copy(u_hbm.at[pl.ds(off, CSLICE)], b_lw.at[csl])
        pltpu.sync_copy(aa_hbm.at[pl.ds(off, CSLICE)], b_msg.at[csl])
        pltpu.sync_copy(ab_hbm.at[pl.ds(off, CSLICE)], b_tmp.at[csl])

        def _add(i, _):
            sl = pl.ds(i * 16, 16)
            b_msg[sl] = b_lw[sl] + b_msg[sl] + b_tmp[sl]
            return 0
        lax.fori_loop(0, CSLICE // 16, _add, 0)

        pltpu.sync_copy(b_msg.at[csl], u_sh.at[pl.ds(off, CSLICE)])

        @pl.when(c == 0)
        def _():
            pltpu.sync_copy(b_msg.at[csl], ucomb_hbm.at[pl.ds(off, CSLICE)])

    def _zero(i, _):
        b_msg[pl.ds(i * 16, 16)] = jnp.zeros((16,), _f32)
        return 0
    lax.fori_loop(0, CSLICE // 16, _zero, 0)
    for j in range(SLICE // CSLICE):
        off = s * SLICE + j * CSLICE
        pltpu.sync_copy(b_msg.at[pl.ds(0, CSLICE)], acc_sh.at[pl.ds(off, CSLICE)])

    plsc.subcore_barrier()

    pltpu.sync_copy(u_sh, u_tab)

    wid = c * 16 + s
    base = wid * EPW

    def _chunk(i, _):
        off = base + i * EC
        pltpu.sync_copy(ei_hbm.at[pl.ds(off, EC)], b_src)
        pltpu.sync_copy(ei_hbm.at[pl.ds(E + off, EC)], b_dst)
        pltpu.sync_copy(lw_hbm.at[pl.ds(off, EC)], b_lw.at[pl.ds(0, EC)])

        def _msg(j, _):
            sl = pl.ds(j * 16, 16)
            uv = plsc.load_gather(u_tab, [b_src[sl]])
            b_msg[sl] = jnp.exp(b_lw[sl]) * uv
            return 0
        lax.fori_loop(0, EC // 16, _msg, 0)

        pltpu.sync_copy(b_msg.at[pl.ds(0, EC)], acc_sh.at[b_dst], add=True)
        return 0
    lax.fori_loop(0, NCH, _chunk, 0)

    plsc.subcore_barrier()

    osl = pl.ds(s * SLICE, SLICE)

    @pl.when(c == 0)
    def _():
        pltpu.sync_copy(acc_sh.at[osl], a0_hbm.at[osl])

    @pl.when(c == 1)
    def _():
        pltpu.sync_copy(acc_sh.at[osl], a1_hbm.at[osl])


_hop = functools.partial(
    pl.kernel,
    out_type=(
        jax.ShapeDtypeStruct((N_PAD,), _f32),
        jax.ShapeDtypeStruct((N_PAD,), _f32),
        jax.ShapeDtypeStruct((N_PAD,), _f32),
    ),
    mesh=plsc.VectorSubcoreMesh(core_axis_name="c", subcore_axis_name="s"),
    compiler_params=pltpu.CompilerParams(needs_layout_passes=False),
    scratch_types=[
        pltpu.VMEM((N_PAD,), _f32),
        pltpu.VMEM((EC,), jnp.int32),
        pltpu.VMEM((EC,), jnp.int32),
        pltpu.VMEM((EC,), _f32),
        pltpu.VMEM((EC,), _f32),
        pltpu.VMEM((EC,), _f32),
        pltpu.VMEM_SHARED((N_PAD,), _f32),
        pltpu.VMEM_SHARED((N_PAD,), _f32),
    ],
)(_hop_body)


def _fin_body(u_hbm, aa_hbm, ab_hbm, out_hbm, b_a, b_b, b_c):
    c = lax.axis_index("c")
    s = lax.axis_index("s")
    wid = c * 16 + s
    for j in range(GSLICE // CSLICE):
        off = wid * GSLICE + j * CSLICE
        pltpu.sync_copy(u_hbm.at[pl.ds(off, CSLICE)], b_a)
        pltpu.sync_copy(aa_hbm.at[pl.ds(off, CSLICE)], b_b)
        pltpu.sync_copy(ab_hbm.at[pl.ds(off, CSLICE)], b_c)

        def _add(i, _):
            sl = pl.ds(i * 16, 16)
            b_a[sl] = b_a[sl] + b_b[sl] + b_c[sl]
            return 0
        lax.fori_loop(0, CSLICE // 16, _add, 0)
        pltpu.sync_copy(b_a, out_hbm.at[pl.ds(off, CSLICE)])


_fin = functools.partial(
    pl.kernel,
    out_type=jax.ShapeDtypeStruct((N_PAD,), _f32),
    mesh=plsc.VectorSubcoreMesh(core_axis_name="c", subcore_axis_name="s"),
    scratch_types=[
        pltpu.VMEM((CSLICE,), _f32),
        pltpu.VMEM((CSLICE,), _f32),
        pltpu.VMEM((CSLICE,), _f32),
    ],
)(_fin_body)


def kernel(xt, edge_index, log_w, B, W0, b0, W1, b1, W2, b2, W3, b3):
    xt_pad = jnp.pad(xt, ((0, N_PAD - N), (0, 0)))
    u = _mlp(xt_pad, B, W0, b0, W1, b1, W2, b2, W3, b3).reshape(N_PAD)
    ei = edge_index.reshape(2 * E)
    z = jnp.zeros((N_PAD,), _f32)
    a0, a1 = z, z
    for _ in range(K):
        u, a0, a1 = _hop(u, a0, a1, ei, log_w)
    u = _fin(u, a0, a1)
    return u[:N].reshape(N, 1)


# speedup vs baseline: 275.4864x; 1.9761x over previous
import functools

import jax
import jax.numpy as jnp
from jax import lax
from jax.experimental import pallas as pl
from jax.experimental.pallas import tpu as pltpu
from jax.experimental.pallas import tpu_sc as plsc

N = 100000
E = 6400000
NF = 64
H = 128
K = 3

N_PAD = 102400
ROWS = 2048
SLICE = N_PAD // 16
CSLICE = SLICE // 4
GSLICE = N_PAD // 32
EPW = E // 32
EC = 2000
NCH = EPW // EC

_f32 = jnp.float32


def _mlp_body(x_ref, b_ref, w0_ref, b0_ref, w1_ref, b1_ref, w2_ref, b2_ref,
              w3_ref, b3_ref, o_ref):
    x = x_ref[...]
    proj = jnp.dot(x, b_ref[...], preferred_element_type=_f32)
    feat = jnp.concatenate([jnp.sin(proj), jnp.cos(proj)], axis=-1)
    h = jnp.tanh(jnp.dot(feat, w0_ref[...], preferred_element_type=_f32)
                 + b0_ref[...])
    h = jnp.tanh(jnp.dot(h, w1_ref[...], preferred_element_type=_f32)
                 + b1_ref[...])
    h = jnp.tanh(jnp.dot(h, w2_ref[...], preferred_element_type=_f32)
                 + b2_ref[...])
    o_ref[...] = jnp.dot(h, w3_ref[...], preferred_element_type=_f32) + b3_ref[...]


def _mlp(xt_pad, B, W0, b0, W1, b1, W2, b2, W3, b3):
    grid = (N_PAD // ROWS,)
    full = lambda r, c: pl.BlockSpec((r, c), lambda i: (0, 0))
    return pl.pallas_call(
        _mlp_body,
        grid=grid,
        in_specs=[
            pl.BlockSpec((ROWS, 2), lambda i: (i, 0)),
            full(2, NF), full(2 * NF, H), full(1, H), full(H, H), full(1, H),
            full(H, H), full(1, H), full(H, 1), full(1, 1),
        ],
        out_specs=pl.BlockSpec((ROWS, 1), lambda i: (i, 0)),
        out_shape=jax.ShapeDtypeStruct((N_PAD, 1), _f32),
    )(xt_pad, B, W0, b0.reshape(1, H), W1, b1.reshape(1, H),
      W2, b2.reshape(1, H), W3, b3.reshape(1, 1))


def _hop_body(u_hbm, aa_hbm, ab_hbm, ei_hbm, lw_hbm,
              ucomb_hbm, a0_hbm, a1_hbm,
              u_tab, b_srcA, b_dstA, b_lwA, b_msgA,
              b_srcB, b_dstB, b_lwB, b_msgB, sh_buf,
              sem_la, sem_lb, sem_sa, sem_sb):
    c = lax.axis_index("c")
    s = lax.axis_index("s")
    wid = c * 16 + s
    base = wid * EPW

    def _loads(off, bs, bd, blw, sem):
        pltpu.async_copy(ei_hbm.at[pl.ds(off, EC)], bs, sem)
        pltpu.async_copy(ei_hbm.at[pl.ds(E + off, EC)], bd, sem)
        pltpu.async_copy(lw_hbm.at[pl.ds(off, EC)], blw, sem)

    def _wait_loads(bs, bd, blw, sem):
        pltpu.make_async_copy(ei_hbm.at[pl.ds(0, EC)], bs, sem).wait()
        pltpu.make_async_copy(ei_hbm.at[pl.ds(0, EC)], bd, sem).wait()
        pltpu.make_async_copy(lw_hbm.at[pl.ds(0, EC)], blw, sem).wait()

    def _compute(bs, blw, bm):
        @plsc.parallel_loop(0, EC // 16, 1, unroll=5)
        def _(j):
            sl = pl.ds(j * 16, 16)
            bm[sl] = jnp.exp(blw[sl]) * plsc.load_gather(u_tab, [bs[sl]])

    def _wait_scatter(bm, bd, sem):
        pltpu.make_async_copy(bm, sh_buf.at[bd], sem).wait()

    for j in range(SLICE // CSLICE):
        off = s * SLICE + j * CSLICE
        csl = pl.ds(0, CSLICE)
        pltpu.sync_copy(u_hbm.at[pl.ds(off, CSLICE)], b_lwA.at[csl])
        pltpu.sync_copy(aa_hbm.at[pl.ds(off, CSLICE)], b_msgA.at[csl])
        pltpu.sync_copy(ab_hbm.at[pl.ds(off, CSLICE)], b_lwB.at[csl])

        def _add(i, _):
            sl = pl.ds(i * 16, 16)
            b_msgA[sl] = b_lwA[sl] + b_msgA[sl] + b_lwB[sl]
            return 0
        lax.fori_loop(0, CSLICE // 16, _add, 0)

        pltpu.sync_copy(b_msgA.at[csl], sh_buf.at[pl.ds(off, CSLICE)])

        @pl.when(c == 0)
        def _():
            pltpu.sync_copy(b_msgA.at[csl], ucomb_hbm.at[pl.ds(off, CSLICE)])

    _loads(base, b_srcA, b_dstA, b_lwA, sem_la)

    plsc.subcore_barrier()

    pltpu.sync_copy(sh_buf, u_tab)

    plsc.subcore_barrier()

    def _zero(i, _):
        b_msgB[pl.ds(i * 16, 16)] = jnp.zeros((16,), _f32)
        return 0
    lax.fori_loop(0, CSLICE // 16, _zero, 0)
    for j in range(SLICE // CSLICE):
        off = s * SLICE + j * CSLICE
        pltpu.sync_copy(b_msgB.at[pl.ds(0, CSLICE)], sh_buf.at[pl.ds(off, CSLICE)])

    plsc.subcore_barrier()

    def _pair(i, _):
        @pl.when(i > 0)
        def _():
            _wait_scatter(b_msgB, b_dstB, sem_sb)
        _loads(base + (2 * i + 1) * EC, b_srcB, b_dstB, b_lwB, sem_lb)
        _wait_loads(b_srcA, b_dstA, b_lwA, sem_la)
        _compute(b_srcA, b_lwA, b_msgA)
        pltpu.async_copy(b_msgA, sh_buf.at[b_dstA], sem_sa, add=True)
        _wait_loads(b_srcB, b_dstB, b_lwB, sem_lb)
        _compute(b_srcB, b_lwB, b_msgB)
        pltpu.async_copy(b_msgB, sh_buf.at[b_dstB], sem_sb, add=True)
        _wait_scatter(b_msgA, b_dstA, sem_sa)

        @pl.when(i < NCH // 2 - 1)
        def _():
            _loads(base + (2 * i + 2) * EC, b_srcA, b_dstA, b_lwA, sem_la)
        return 0
    lax.fori_loop(0, NCH // 2, _pair, 0)
    _wait_scatter(b_msgB, b_dstB, sem_sb)

    plsc.subcore_barrier()

    osl = pl.ds(s * SLICE, SLICE)

    @pl.when(c == 0)
    def _():
        pltpu.sync_copy(sh_buf.at[osl], a0_hbm.at[osl])

    @pl.when(c == 1)
    def _():
        pltpu.sync_copy(sh_buf.at[osl], a1_hbm.at[osl])


_hop = functools.partial(
    pl.kernel,
    out_type=(
        jax.ShapeDtypeStruct((N_PAD,), _f32),
        jax.ShapeDtypeStruct((N_PAD,), _f32),
        jax.ShapeDtypeStruct((N_PAD,), _f32),
    ),
    mesh=plsc.VectorSubcoreMesh(core_axis_name="c", subcore_axis_name="s"),
    compiler_params=pltpu.CompilerParams(needs_layout_passes=False),
    scratch_types=[
        pltpu.VMEM((N_PAD,), _f32),
        pltpu.VMEM((EC,), jnp.int32),
        pltpu.VMEM((EC,), jnp.int32),
        pltpu.VMEM((EC,), _f32),
        pltpu.VMEM((EC,), _f32),
        pltpu.VMEM((EC,), jnp.int32),
        pltpu.VMEM((EC,), jnp.int32),
        pltpu.VMEM((EC,), _f32),
        pltpu.VMEM((EC,), _f32),
        pltpu.VMEM_SHARED((N_PAD,), _f32),
        pltpu.SemaphoreType.DMA,
        pltpu.SemaphoreType.DMA,
        pltpu.SemaphoreType.DMA,
        pltpu.SemaphoreType.DMA,
    ],
)(_hop_body)


def _fin_body(u_hbm, aa_hbm, ab_hbm, out_hbm, b_a, b_b, b_c):
    c = lax.axis_index("c")
    s = lax.axis_index("s")
    wid = c * 16 + s
    for j in range(GSLICE // CSLICE):
        off = wid * GSLICE + j * CSLICE
        pltpu.sync_copy(u_hbm.at[pl.ds(off, CSLICE)], b_a)
        pltpu.sync_copy(aa_hbm.at[pl.ds(off, CSLICE)], b_b)
        pltpu.sync_copy(ab_hbm.at[pl.ds(off, CSLICE)], b_c)

        def _add(i, _):
            sl = pl.ds(i * 16, 16)
            b_a[sl] = b_a[sl] + b_b[sl] + b_c[sl]
            return 0
        lax.fori_loop(0, CSLICE // 16, _add, 0)
        pltpu.sync_copy(b_a, out_hbm.at[pl.ds(off, CSLICE)])


_fin = functools.partial(
    pl.kernel,
    out_type=jax.ShapeDtypeStruct((N_PAD,), _f32),
    mesh=plsc.VectorSubcoreMesh(core_axis_name="c", subcore_axis_name="s"),
    scratch_types=[
        pltpu.VMEM((CSLICE,), _f32),
        pltpu.VMEM((CSLICE,), _f32),
        pltpu.VMEM((CSLICE,), _f32),
    ],
)(_fin_body)


def kernel(xt, edge_index, log_w, B, W0, b0, W1, b1, W2, b2, W3, b3):
    xt_pad = jnp.pad(xt, ((0, N_PAD - N), (0, 0)))
    u = _mlp(xt_pad, B, W0, b0, W1, b1, W2, b2, W3, b3).reshape(N_PAD)
    ei = edge_index.reshape(2 * E)
    z = jnp.zeros((N_PAD,), _f32)
    a0, a1 = z, z
    for _ in range(K):
        u, a0, a1 = _hop(u, a0, a1, ei, log_w)
    u = _fin(u, a0, a1)
    return u[:N].reshape(N, 1)


# speedup vs baseline: 374.7428x; 1.3603x over previous
import functools

import jax
import jax.numpy as jnp
from jax import lax
from jax.experimental import pallas as pl
from jax.experimental.pallas import tpu as pltpu
from jax.experimental.pallas import tpu_sc as plsc

N = 100000
E = 6400000
NF = 64
H = 128
K = 3

N_PAD = 100352
ROWS = 2048
SLICE = N_PAD // 16
CSLICE = SLICE // 4
GSLICE = N_PAD // 32
EPW = E // 32
EC = 2000
NCH = EPW // EC

_f32 = jnp.float32


def _mlp_body(x_ref, bT_ref, w0T_ref, b0_ref, w1T_ref, b1_ref, w2T_ref,
              b2_ref, w3T_ref, b3_ref, o_ref):
    x = x_ref[...]
    projT = jnp.dot(bT_ref[...], x, preferred_element_type=_f32)
    featT = jnp.concatenate([jnp.sin(projT), jnp.cos(projT)], axis=0)
    h = jnp.tanh(jnp.dot(w0T_ref[...], featT, preferred_element_type=_f32)
                 + b0_ref[...])
    h = jnp.tanh(jnp.dot(w1T_ref[...], h, preferred_element_type=_f32)
                 + b1_ref[...])
    h = jnp.tanh(jnp.dot(w2T_ref[...], h, preferred_element_type=_f32)
                 + b2_ref[...])
    o_ref[...] = (jnp.dot(w3T_ref[...], h, preferred_element_type=_f32)
                  + b3_ref[...])


def _mlp(xtT, B, W0, b0, W1, b1, W2, b2, W3, b3):
    grid = (N_PAD // ROWS,)
    full = lambda r, c: pl.BlockSpec((r, c), lambda i: (0, 0))
    return pl.pallas_call(
        _mlp_body,
        grid=grid,
        in_specs=[
            pl.BlockSpec((2, ROWS), lambda i: (0, i)),
            full(NF, 2), full(H, 2 * NF), full(H, 1), full(H, H), full(H, 1),
            full(H, H), full(H, 1), full(1, H), full(1, 1),
        ],
        out_specs=pl.BlockSpec((1, ROWS), lambda i: (0, i)),
        out_shape=jax.ShapeDtypeStruct((1, N_PAD), _f32),
    )(xtT, B.T, W0.T, b0.reshape(H, 1), W1.T, b1.reshape(H, 1),
      W2.T, b2.reshape(H, 1), W3.T, b3.reshape(1, 1))


def _hop_body(u_hbm, aa_hbm, ab_hbm, ei_hbm, lw_hbm,
              ucomb_hbm, a0_hbm, a1_hbm,
              u_tab, b_srcA, b_dstA, b_lwA, b_msgA,
              b_srcB, b_dstB, b_lwB, b_msgB, sh_buf,
              sem_la, sem_lb, sem_sa, sem_sb):
    c = lax.axis_index("c")
    s = lax.axis_index("s")
    wid = c * 16 + s
    base = wid * EPW

    def _loads(off, bs, bd, blw, sem):
        pltpu.async_copy(ei_hbm.at[pl.ds(off, EC)], bs, sem)
        pltpu.async_copy(ei_hbm.at[pl.ds(E + off, EC)], bd, sem)
        pltpu.async_copy(lw_hbm.at[pl.ds(off, EC)], blw, sem)

    def _wait_loads(bs, bd, blw, sem):
        pltpu.make_async_copy(ei_hbm.at[pl.ds(0, EC)], bs, sem).wait()
        pltpu.make_async_copy(ei_hbm.at[pl.ds(0, EC)], bd, sem).wait()
        pltpu.make_async_copy(lw_hbm.at[pl.ds(0, EC)], blw, sem).wait()

    def _compute(bs, blw, bm):
        @plsc.parallel_loop(0, EC // 16, 1, unroll=5)
        def _(j):
            sl = pl.ds(j * 16, 16)
            bm[sl] = jnp.exp(blw[sl]) * plsc.load_gather(u_tab, [bs[sl]])

    def _wait_scatter(bm, bd, sem):
        pltpu.make_async_copy(bm, sh_buf.at[bd], sem).wait()

    for j in range(SLICE // CSLICE):
        off = s * SLICE + j * CSLICE
        csl = pl.ds(0, CSLICE)
        pltpu.sync_copy(u_hbm.at[pl.ds(off, CSLICE)], b_lwA.at[csl])
        pltpu.sync_copy(aa_hbm.at[pl.ds(off, CSLICE)], b_msgA.at[csl])
        pltpu.sync_copy(ab_hbm.at[pl.ds(off, CSLICE)], b_lwB.at[csl])

        def _add(i, _):
            sl = pl.ds(i * 16, 16)
            b_msgA[sl] = b_lwA[sl] + b_msgA[sl] + b_lwB[sl]
            return 0
        lax.fori_loop(0, CSLICE // 16, _add, 0)

        pltpu.sync_copy(b_msgA.at[csl], sh_buf.at[pl.ds(off, CSLICE)])

        @pl.when(c == 0)
        def _():
            pltpu.sync_copy(b_msgA.at[csl], ucomb_hbm.at[pl.ds(off, CSLICE)])

    _loads(base, b_srcA, b_dstA, b_lwA, sem_la)

    plsc.subcore_barrier()

    pltpu.sync_copy(sh_buf, u_tab)

    plsc.subcore_barrier()

    def _zero(i, _):
        b_msgB[pl.ds(i * 16, 16)] = jnp.zeros((16,), _f32)
        return 0
    lax.fori_loop(0, CSLICE // 16, _zero, 0)
    for j in range(SLICE // CSLICE):
        off = s * SLICE + j * CSLICE
        pltpu.sync_copy(b_msgB.at[pl.ds(0, CSLICE)], sh_buf.at[pl.ds(off, CSLICE)])

    plsc.subcore_barrier()

    def _pair(i, _):
        @pl.when(i > 0)
        def _():
            _wait_scatter(b_msgB, b_dstB, sem_sb)
        _loads(base + (2 * i + 1) * EC, b_srcB, b_dstB, b_lwB, sem_lb)
        _wait_loads(b_srcA, b_dstA, b_lwA, sem_la)
        _compute(b_srcA, b_lwA, b_msgA)
        pltpu.async_copy(b_msgA, sh_buf.at[b_dstA], sem_sa, add=True)
        _wait_loads(b_srcB, b_dstB, b_lwB, sem_lb)
        _compute(b_srcB, b_lwB, b_msgB)
        pltpu.async_copy(b_msgB, sh_buf.at[b_dstB], sem_sb, add=True)
        _wait_scatter(b_msgA, b_dstA, sem_sa)

        @pl.when(i < NCH // 2 - 1)
        def _():
            _loads(base + (2 * i + 2) * EC, b_srcA, b_dstA, b_lwA, sem_la)
        return 0
    lax.fori_loop(0, NCH // 2, _pair, 0)
    _wait_scatter(b_msgB, b_dstB, sem_sb)

    plsc.subcore_barrier()

    osl = pl.ds(s * SLICE, SLICE)

    @pl.when(c == 0)
    def _():
        pltpu.sync_copy(sh_buf.at[osl], a0_hbm.at[osl])

    @pl.when(c == 1)
    def _():
        pltpu.sync_copy(sh_buf.at[osl], a1_hbm.at[osl])


_hop = functools.partial(
    pl.kernel,
    out_type=(
        jax.ShapeDtypeStruct((N_PAD,), _f32),
        jax.ShapeDtypeStruct((N_PAD,), _f32),
        jax.ShapeDtypeStruct((N_PAD,), _f32),
    ),
    mesh=plsc.VectorSubcoreMesh(core_axis_name="c", subcore_axis_name="s"),
    compiler_params=pltpu.CompilerParams(needs_layout_passes=False),
    scratch_types=[
        pltpu.VMEM((N_PAD,), _f32),
        pltpu.VMEM((EC,), jnp.int32),
        pltpu.VMEM((EC,), jnp.int32),
        pltpu.VMEM((EC,), _f32),
        pltpu.VMEM((EC,), _f32),
        pltpu.VMEM((EC,), jnp.int32),
        pltpu.VMEM((EC,), jnp.int32),
        pltpu.VMEM((EC,), _f32),
        pltpu.VMEM((EC,), _f32),
        pltpu.VMEM_SHARED((N_PAD,), _f32),
        pltpu.SemaphoreType.DMA,
        pltpu.SemaphoreType.DMA,
        pltpu.SemaphoreType.DMA,
        pltpu.SemaphoreType.DMA,
    ],
)(_hop_body)


def _fin_body(u_hbm, aa_hbm, ab_hbm, out_hbm, b_a, b_b, b_c):
    c = lax.axis_index("c")
    s = lax.axis_index("s")
    wid = c * 16 + s
    for j in range(GSLICE // CSLICE):
        off = wid * GSLICE + j * CSLICE
        pltpu.sync_copy(u_hbm.at[pl.ds(off, CSLICE)], b_a)
        pltpu.sync_copy(aa_hbm.at[pl.ds(off, CSLICE)], b_b)
        pltpu.sync_copy(ab_hbm.at[pl.ds(off, CSLICE)], b_c)

        def _add(i, _):
            sl = pl.ds(i * 16, 16)
            b_a[sl] = b_a[sl] + b_b[sl] + b_c[sl]
            return 0
        lax.fori_loop(0, CSLICE // 16, _add, 0)
        pltpu.sync_copy(b_a, out_hbm.at[pl.ds(off, CSLICE)])


_fin = functools.partial(
    pl.kernel,
    out_type=jax.ShapeDtypeStruct((N_PAD,), _f32),
    mesh=plsc.VectorSubcoreMesh(core_axis_name="c", subcore_axis_name="s"),
    scratch_types=[
        pltpu.VMEM((CSLICE,), _f32),
        pltpu.VMEM((CSLICE,), _f32),
        pltpu.VMEM((CSLICE,), _f32),
    ],
)(_fin_body)


def kernel(xt, edge_index, log_w, B, W0, b0, W1, b1, W2, b2, W3, b3):
    u = _mlp(xt.T, B, W0, b0, W1, b1, W2, b2, W3, b3).reshape(N_PAD)
    ei = edge_index.reshape(2 * E)
    z = jnp.zeros((N_PAD,), _f32)
    a0, a1 = z, z
    for _ in range(K):
        u, a0, a1 = _hop(u, a0, a1, ei, log_w)
    u = _fin(u, a0, a1)
    return u[:N].reshape(N, 1)


# speedup vs baseline: 428.9305x; 1.1446x over previous
import functools

import jax
import jax.numpy as jnp
from jax import lax
from jax.experimental import pallas as pl
from jax.experimental.pallas import tpu as pltpu
from jax.experimental.pallas import tpu_sc as plsc

N = 100000
E = 6400000
NF = 64
H = 128
K = 3

N_PAD = 100352
ROWS = 2048
SLICE = N_PAD // 16
CSLICE = SLICE // 4
GSLICE = N_PAD // 32
EPW = E // 32
EC = 2000
NCH = EPW // EC

_f32 = jnp.float32


_INV2PI = 0.15915493667125702
_TWOPI = 6.2831854820251465
_SC = (0.9999999378055222, -0.16666621099119006, 0.008332791415928265,
       -0.00019817628169256163, 2.7088274845432636e-06, -2.0697969171923347e-08)
_CC = (0.9999992105908722, -0.499994212222991, 0.04165977670694633,
       -0.001385878635320515, 2.4202894902042305e-05, -2.1972754565835473e-07)


def _sincos(x):
    q = jnp.floor(x * _INV2PI + 0.5)
    r = x - q * _TWOPI
    t = r * r
    s = _SC[0] + t * (_SC[1] + t * (_SC[2] + t * (_SC[3] + t * (_SC[4] + t * _SC[5]))))
    c = _CC[0] + t * (_CC[1] + t * (_CC[2] + t * (_CC[3] + t * (_CC[4] + t * _CC[5]))))
    return s * r, c


def _mlp_body(x_ref, bT_ref, w0T_ref, b0_ref, w1T_ref, b1_ref, w2T_ref,
              b2_ref, w3T_ref, b3_ref, o_ref):
    x = x_ref[...]
    projT = jnp.dot(bT_ref[...], x, preferred_element_type=_f32)
    sn, cn = _sincos(projT)
    featT = jnp.concatenate([sn, cn], axis=0)
    h = jnp.tanh(jnp.dot(w0T_ref[...], featT, preferred_element_type=_f32)
                 + b0_ref[...])
    h = jnp.tanh(jnp.dot(w1T_ref[...], h, preferred_element_type=_f32)
                 + b1_ref[...])
    h = jnp.tanh(jnp.dot(w2T_ref[...], h, preferred_element_type=_f32)
                 + b2_ref[...])
    o_ref[...] = (jnp.dot(w3T_ref[...], h, preferred_element_type=_f32)
                  + b3_ref[...])


def _mlp(xtT, B, W0, b0, W1, b1, W2, b2, W3, b3):
    grid = (N_PAD // ROWS,)
    full = lambda r, c: pl.BlockSpec((r, c), lambda i: (0, 0))
    return pl.pallas_call(
        _mlp_body,
        grid=grid,
        in_specs=[
            pl.BlockSpec((2, ROWS), lambda i: (0, i)),
            full(NF, 2), full(H, 2 * NF), full(H, 1), full(H, H), full(H, 1),
            full(H, H), full(H, 1), full(1, H), full(1, 1),
        ],
        out_specs=pl.BlockSpec((1, ROWS), lambda i: (0, i)),
        out_shape=jax.ShapeDtypeStruct((1, N_PAD), _f32),
    )(xtT, B.T, W0.T, b0.reshape(H, 1), W1.T, b1.reshape(H, 1),
      W2.T, b2.reshape(H, 1), W3.T, b3.reshape(1, 1))


def _hop_body(u_hbm, aa_hbm, ab_hbm, ei_hbm, lw_hbm,
              ucomb_hbm, a0_hbm, a1_hbm,
              u_tab, b_srcA, b_dstA, b_lwA, b_msgA,
              b_srcB, b_dstB, b_lwB, b_msgB, sh_buf,
              sem_la, sem_lb, sem_sa, sem_sb):
    c = lax.axis_index("c")
    s = lax.axis_index("s")
    wid = c * 16 + s
    base = wid * EPW

    def _loads(off, bs, bd, blw, sem):
        pltpu.async_copy(ei_hbm.at[pl.ds(off, EC)], bs, sem)
        pltpu.async_copy(ei_hbm.at[pl.ds(E + off, EC)], bd, sem)
        pltpu.async_copy(lw_hbm.at[pl.ds(off, EC)], blw, sem)

    def _wait_loads(bs, bd, blw, sem):
        pltpu.make_async_copy(ei_hbm.at[pl.ds(0, EC)], bs, sem).wait()
        pltpu.make_async_copy(ei_hbm.at[pl.ds(0, EC)], bd, sem).wait()
        pltpu.make_async_copy(lw_hbm.at[pl.ds(0, EC)], blw, sem).wait()

    def _compute(bs, blw, bm):
        @plsc.parallel_loop(0, EC // 16, 1, unroll=5)
        def _(j):
            sl = pl.ds(j * 16, 16)
            bm[sl] = jnp.exp(blw[sl]) * plsc.load_gather(u_tab, [bs[sl]])

    def _wait_scatter(bm, bd, sem):
        pltpu.make_async_copy(bm, sh_buf.at[bd], sem).wait()

    for j in range(SLICE // CSLICE):
        off = s * SLICE + j * CSLICE
        csl = pl.ds(0, CSLICE)
        pltpu.sync_copy(u_hbm.at[pl.ds(off, CSLICE)], b_lwA.at[csl])
        pltpu.sync_copy(aa_hbm.at[pl.ds(off, CSLICE)], b_msgA.at[csl])
        pltpu.sync_copy(ab_hbm.at[pl.ds(off, CSLICE)], b_lwB.at[csl])

        def _add(i, _):
            sl = pl.ds(i * 16, 16)
            b_msgA[sl] = b_lwA[sl] + b_msgA[sl] + b_lwB[sl]
            return 0
        lax.fori_loop(0, CSLICE // 16, _add, 0)

        pltpu.sync_copy(b_msgA.at[csl], sh_buf.at[pl.ds(off, CSLICE)])

        @pl.when(c == 0)
        def _():
            pltpu.sync_copy(b_msgA.at[csl], ucomb_hbm.at[pl.ds(off, CSLICE)])

    _loads(base, b_srcA, b_dstA, b_lwA, sem_la)

    plsc.subcore_barrier()

    pltpu.sync_copy(sh_buf, u_tab)

    plsc.subcore_barrier()

    def _zero(i, _):
        b_msgB[pl.ds(i * 16, 16)] = jnp.zeros((16,), _f32)
        return 0
    lax.fori_loop(0, CSLICE // 16, _zero, 0)
    for j in range(SLICE // CSLICE):
        off = s * SLICE + j * CSLICE
        pltpu.sync_copy(b_msgB.at[pl.ds(0, CSLICE)], sh_buf.at[pl.ds(off, CSLICE)])

    plsc.subcore_barrier()

    def _pair(i, _):
        @pl.when(i > 0)
        def _():
            _wait_scatter(b_msgB, b_dstB, sem_sb)
        _loads(base + (2 * i + 1) * EC, b_srcB, b_dstB, b_lwB, sem_lb)
        _wait_loads(b_srcA, b_dstA, b_lwA, sem_la)
        _compute(b_srcA, b_lwA, b_msgA)
        pltpu.async_copy(b_msgA, sh_buf.at[b_dstA], sem_sa, add=True)
        _wait_loads(b_srcB, b_dstB, b_lwB, sem_lb)
        _compute(b_srcB, b_lwB, b_msgB)
        pltpu.async_copy(b_msgB, sh_buf.at[b_dstB], sem_sb, add=True)
        _wait_scatter(b_msgA, b_dstA, sem_sa)

        @pl.when(i < NCH // 2 - 1)
        def _():
            _loads(base + (2 * i + 2) * EC, b_srcA, b_dstA, b_lwA, sem_la)
        return 0
    lax.fori_loop(0, NCH // 2, _pair, 0)
    _wait_scatter(b_msgB, b_dstB, sem_sb)

    plsc.subcore_barrier()

    osl = pl.ds(s * SLICE, SLICE)

    @pl.when(c == 0)
    def _():
        pltpu.sync_copy(sh_buf.at[osl], a0_hbm.at[osl])

    @pl.when(c == 1)
    def _():
        pltpu.sync_copy(sh_buf.at[osl], a1_hbm.at[osl])


_hop = functools.partial(
    pl.kernel,
    out_type=(
        jax.ShapeDtypeStruct((N_PAD,), _f32),
        jax.ShapeDtypeStruct((N_PAD,), _f32),
        jax.ShapeDtypeStruct((N_PAD,), _f32),
    ),
    mesh=plsc.VectorSubcoreMesh(core_axis_name="c", subcore_axis_name="s"),
    compiler_params=pltpu.CompilerParams(needs_layout_passes=False),
    scratch_types=[
        pltpu.VMEM((N_PAD,), _f32),
        pltpu.VMEM((EC,), jnp.int32),
        pltpu.VMEM((EC,), jnp.int32),
        pltpu.VMEM((EC,), _f32),
        pltpu.VMEM((EC,), _f32),
        pltpu.VMEM((EC,), jnp.int32),
        pltpu.VMEM((EC,), jnp.int32),
        pltpu.VMEM((EC,), _f32),
        pltpu.VMEM((EC,), _f32),
        pltpu.VMEM_SHARED((N_PAD,), _f32),
        pltpu.SemaphoreType.DMA,
        pltpu.SemaphoreType.DMA,
        pltpu.SemaphoreType.DMA,
        pltpu.SemaphoreType.DMA,
    ],
)(_hop_body)


def _fin_body(u_hbm, aa_hbm, ab_hbm, out_hbm, b_a, b_b, b_c):
    c = lax.axis_index("c")
    s = lax.axis_index("s")
    wid = c * 16 + s
    for j in range(GSLICE // CSLICE):
        off = wid * GSLICE + j * CSLICE
        pltpu.sync_copy(u_hbm.at[pl.ds(off, CSLICE)], b_a)
        pltpu.sync_copy(aa_hbm.at[pl.ds(off, CSLICE)], b_b)
        pltpu.sync_copy(ab_hbm.at[pl.ds(off, CSLICE)], b_c)

        def _add(i, _):
            sl = pl.ds(i * 16, 16)
            b_a[sl] = b_a[sl] + b_b[sl] + b_c[sl]
            return 0
        lax.fori_loop(0, CSLICE // 16, _add, 0)
        pltpu.sync_copy(b_a, out_hbm.at[pl.ds(off, CSLICE)])


_fin = functools.partial(
    pl.kernel,
    out_type=jax.ShapeDtypeStruct((N_PAD,), _f32),
    mesh=plsc.VectorSubcoreMesh(core_axis_name="c", subcore_axis_name="s"),
    scratch_types=[
        pltpu.VMEM((CSLICE,), _f32),
        pltpu.VMEM((CSLICE,), _f32),
        pltpu.VMEM((CSLICE,), _f32),
    ],
)(_fin_body)


def kernel(xt, edge_index, log_w, B, W0, b0, W1, b1, W2, b2, W3, b3):
    u = _mlp(xt.T, B, W0, b0, W1, b1, W2, b2, W3, b3).reshape(N_PAD)
    ei = edge_index.reshape(2 * E)
    z = jnp.zeros((N_PAD,), _f32)
    a0, a1 = z, z
    for _ in range(K):
        u, a0, a1 = _hop(u, a0, a1, ei, log_w)
    u = _fin(u, a0, a1)
    return u[:N].reshape(N, 1)


# speedup vs baseline: 505.4142x; 1.1783x over previous
import functools

import jax
import jax.numpy as jnp
from jax import lax
from jax.experimental import pallas as pl
from jax.experimental.pallas import tpu as pltpu
from jax.experimental.pallas import tpu_sc as plsc

N = 100000
E = 6400000
NF = 64
H = 128
K = 3

N_PAD = 100352
ROWS = 2048
SLICE = N_PAD // 16
CSLICE = SLICE // 4
GSLICE = N_PAD // 32
EPW = E // 32
EC = 4000
NCH = EPW // EC
N_TAB = N

_f32 = jnp.float32


_INV2PI = 0.15915493667125702
_TWOPI = 6.2831854820251465
_SC = (0.9999999378055222, -0.16666621099119006, 0.008332791415928265,
       -0.00019817628169256163, 2.7088274845432636e-06, -2.0697969171923347e-08)
_CC = (0.9999992105908722, -0.499994212222991, 0.04165977670694633,
       -0.001385878635320515, 2.4202894902042305e-05, -2.1972754565835473e-07)


def _sincos(x):
    q = jnp.floor(x * _INV2PI + 0.5)
    r = x - q * _TWOPI
    t = r * r
    s = _SC[0] + t * (_SC[1] + t * (_SC[2] + t * (_SC[3] + t * (_SC[4] + t * _SC[5]))))
    c = _CC[0] + t * (_CC[1] + t * (_CC[2] + t * (_CC[3] + t * (_CC[4] + t * _CC[5]))))
    return s * r, c


def _mlp_body(x_ref, bT_ref, w0T_ref, b0_ref, w1T_ref, b1_ref, w2T_ref,
              b2_ref, w3T_ref, b3_ref, o_ref):
    x = x_ref[...]
    projT = jnp.dot(bT_ref[...], x, preferred_element_type=_f32)
    sn, cn = _sincos(projT)
    featT = jnp.concatenate([sn, cn], axis=0)
    h = jnp.tanh(jnp.dot(w0T_ref[...], featT, preferred_element_type=_f32)
                 + b0_ref[...])
    h = jnp.tanh(jnp.dot(w1T_ref[...], h, preferred_element_type=_f32)
                 + b1_ref[...])
    h = jnp.tanh(jnp.dot(w2T_ref[...], h, preferred_element_type=_f32)
                 + b2_ref[...])
    o_ref[...] = (jnp.dot(w3T_ref[...], h, preferred_element_type=_f32)
                  + b3_ref[...])


def _mlp(xtT, B, W0, b0, W1, b1, W2, b2, W3, b3):
    grid = (N_PAD // ROWS,)
    full = lambda r, c: pl.BlockSpec((r, c), lambda i: (0, 0))
    return pl.pallas_call(
        _mlp_body,
        grid=grid,
        in_specs=[
            pl.BlockSpec((2, ROWS), lambda i: (0, i)),
            full(NF, 2), full(H, 2 * NF), full(H, 1), full(H, H), full(H, 1),
            full(H, H), full(H, 1), full(1, H), full(1, 1),
        ],
        out_specs=pl.BlockSpec((1, ROWS), lambda i: (0, i)),
        out_shape=jax.ShapeDtypeStruct((1, N_PAD), _f32),
    )(xtT, B.T, W0.T, b0.reshape(H, 1), W1.T, b1.reshape(H, 1),
      W2.T, b2.reshape(H, 1), W3.T, b3.reshape(1, 1))


def _hop_body(aa_hbm, ab_hbm, ei_hbm, lw_hbm, a0_hbm, a1_hbm,
              u_tab, b_srcA, b_dstA, b_lwA, b_srcB, b_dstB, b_lwB, sh_buf,
              sem_la, sem_lb, sem_sa, sem_sb):
    c = lax.axis_index("c")
    s = lax.axis_index("s")
    wid = c * 16 + s
    base = wid * EPW

    def _loads(off, bs, bd, blw, sem):
        pltpu.async_copy(ei_hbm.at[pl.ds(off, EC)], bs, sem)
        pltpu.async_copy(ei_hbm.at[pl.ds(E + off, EC)], bd, sem)
        pltpu.async_copy(lw_hbm.at[pl.ds(off, EC)], blw, sem)

    def _wait_loads(bs, bd, blw, sem):
        pltpu.make_async_copy(ei_hbm.at[pl.ds(0, EC)], bs, sem).wait()
        pltpu.make_async_copy(ei_hbm.at[pl.ds(0, EC)], bd, sem).wait()
        pltpu.make_async_copy(lw_hbm.at[pl.ds(0, EC)], blw, sem).wait()

    def _compute(bs, blw):
        @plsc.parallel_loop(0, EC // 16, 1, unroll=10)
        def _(j):
            sl = pl.ds(j * 16, 16)
            blw[sl] = jnp.exp(blw[sl]) * plsc.load_gather(u_tab, [bs[sl]])

    def _wait_scatter(blw, bd, sem):
        pltpu.make_async_copy(blw, sh_buf.at[bd], sem).wait()

    for j in range(SLICE // CSLICE):
        off = s * SLICE + j * CSLICE
        csl = pl.ds(0, CSLICE)
        pltpu.sync_copy(aa_hbm.at[pl.ds(off, CSLICE)], b_lwA.at[csl])
        pltpu.sync_copy(ab_hbm.at[pl.ds(off, CSLICE)], b_lwB.at[csl])

        def _add(i, _):
            sl = pl.ds(i * 16, 16)
            b_lwA[sl] = b_lwA[sl] + b_lwB[sl]
            return 0
        lax.fori_loop(0, CSLICE // 16, _add, 0)

        pltpu.sync_copy(b_lwA.at[csl], sh_buf.at[pl.ds(off, CSLICE)])

    _loads(base, b_srcA, b_dstA, b_lwA, sem_la)

    plsc.subcore_barrier()

    pltpu.sync_copy(sh_buf.at[pl.ds(0, N_TAB)], u_tab)

    plsc.subcore_barrier()

    @pl.when(c == 1)
    def _():
        def _zfill(i, _):
            b_lwB[pl.ds(i * 16, 16)] = jnp.zeros((16,), _f32)
            return 0
        lax.fori_loop(0, CSLICE // 16, _zfill, 0)
        for j in range(SLICE // CSLICE):
            off = s * SLICE + j * CSLICE
            pltpu.sync_copy(b_lwB.at[pl.ds(0, CSLICE)],
                            sh_buf.at[pl.ds(off, CSLICE)])

    plsc.subcore_barrier()

    def _pair(i, _):
        @pl.when(i > 0)
        def _():
            _wait_scatter(b_lwB, b_dstB, sem_sb)
        _loads(base + (2 * i + 1) * EC, b_srcB, b_dstB, b_lwB, sem_lb)
        _wait_loads(b_srcA, b_dstA, b_lwA, sem_la)
        _compute(b_srcA, b_lwA)
        pltpu.async_copy(b_lwA, sh_buf.at[b_dstA], sem_sa, add=True)
        _wait_loads(b_srcB, b_dstB, b_lwB, sem_lb)
        _compute(b_srcB, b_lwB)
        pltpu.async_copy(b_lwB, sh_buf.at[b_dstB], sem_sb, add=True)
        _wait_scatter(b_lwA, b_dstA, sem_sa)

        @pl.when(i < NCH // 2 - 1)
        def _():
            _loads(base + (2 * i + 2) * EC, b_srcA, b_dstA, b_lwA, sem_la)
        return 0
    lax.fori_loop(0, NCH // 2, _pair, 0)
    _wait_scatter(b_lwB, b_dstB, sem_sb)

    plsc.subcore_barrier()

    osl = pl.ds(s * SLICE, SLICE)

    @pl.when(c == 0)
    def _():
        pltpu.sync_copy(sh_buf.at[osl], a0_hbm.at[osl])

    @pl.when(c == 1)
    def _():
        pltpu.sync_copy(sh_buf.at[osl], a1_hbm.at[osl])


_hop = functools.partial(
    pl.kernel,
    out_type=(
        jax.ShapeDtypeStruct((N_PAD,), _f32),
        jax.ShapeDtypeStruct((N_PAD,), _f32),
    ),
    mesh=plsc.VectorSubcoreMesh(core_axis_name="c", subcore_axis_name="s"),
    compiler_params=pltpu.CompilerParams(needs_layout_passes=False),
    scratch_types=[
        pltpu.VMEM((N_TAB,), _f32),
        pltpu.VMEM((EC,), jnp.int32),
        pltpu.VMEM((EC,), jnp.int32),
        pltpu.VMEM((EC,), _f32),
        pltpu.VMEM((EC,), jnp.int32),
        pltpu.VMEM((EC,), jnp.int32),
        pltpu.VMEM((EC,), _f32),
        pltpu.VMEM_SHARED((N_PAD,), _f32),
        pltpu.SemaphoreType.DMA,
        pltpu.SemaphoreType.DMA,
        pltpu.SemaphoreType.DMA,
        pltpu.SemaphoreType.DMA,
    ],
)(_hop_body)


def _fin_body(aa_hbm, ab_hbm, out_hbm, b_a, b_b):
    c = lax.axis_index("c")
    s = lax.axis_index("s")
    wid = c * 16 + s
    for j in range(GSLICE // CSLICE):
        off = wid * GSLICE + j * CSLICE
        pltpu.sync_copy(aa_hbm.at[pl.ds(off, CSLICE)], b_a)
        pltpu.sync_copy(ab_hbm.at[pl.ds(off, CSLICE)], b_b)

        def _add(i, _):
            sl = pl.ds(i * 16, 16)
            b_a[sl] = b_a[sl] + b_b[sl]
            return 0
        lax.fori_loop(0, CSLICE // 16, _add, 0)
        pltpu.sync_copy(b_a, out_hbm.at[pl.ds(off, CSLICE)])


_fin = functools.partial(
    pl.kernel,
    out_type=jax.ShapeDtypeStruct((N_PAD,), _f32),
    mesh=plsc.VectorSubcoreMesh(core_axis_name="c", subcore_axis_name="s"),
    scratch_types=[
        pltpu.VMEM((CSLICE,), _f32),
        pltpu.VMEM((CSLICE,), _f32),
    ],
)(_fin_body)


def kernel(xt, edge_index, log_w, B, W0, b0, W1, b1, W2, b2, W3, b3):
    u = _mlp(xt.T, B, W0, b0, W1, b1, W2, b2, W3, b3).reshape(N_PAD)
    ei = edge_index.reshape(2 * E)
    a0, a1 = u, jnp.zeros((N_PAD,), _f32)
    for _ in range(K):
        a0, a1 = _hop(a0, a1, ei, log_w)
    u = _fin(a0, a1)
    return u[:N].reshape(N, 1)


# speedup vs baseline: 506.6755x; 1.0025x over previous
import functools

import jax
import jax.numpy as jnp
from jax import lax
from jax.experimental import pallas as pl
from jax.experimental.pallas import tpu as pltpu
from jax.experimental.pallas import tpu_sc as plsc

N = 100000
E = 6400000
NF = 64
H = 128
K = 3

N_PAD = 100352
ROWS = 2048
SLICE = N_PAD // 16
CSLICE = SLICE // 4
GSLICE = N_PAD // 32
EPW = E // 32
EC = 4000
NCH = EPW // EC
N_TAB = N

_f32 = jnp.float32


_INV2PI = 0.15915493667125702
_TWOPI = 6.2831854820251465
_SC = (0.9999999378055222, -0.16666621099119006, 0.008332791415928265,
       -0.00019817628169256163, 2.7088274845432636e-06, -2.0697969171923347e-08)
_CC = (0.9999992105908722, -0.499994212222991, 0.04165977670694633,
       -0.001385878635320515, 2.4202894902042305e-05, -2.1972754565835473e-07)


def _sincos(x):
    q = jnp.floor(x * _INV2PI + 0.5)
    r = x - q * _TWOPI
    t = r * r
    s = _SC[0] + t * (_SC[1] + t * (_SC[2] + t * (_SC[3] + t * (_SC[4] + t * _SC[5]))))
    c = _CC[0] + t * (_CC[1] + t * (_CC[2] + t * (_CC[3] + t * (_CC[4] + t * _CC[5]))))
    return s * r, c


def _dotT(w_ref, x):
    return lax.dot_general(w_ref[...], x, (((0,), (0,)), ((), ())),
                           preferred_element_type=_f32)


def _mlp_body(x_ref, b_ref, w0_ref, b0_ref, w1_ref, b1_ref, w2_ref,
              b2_ref, w3_ref, b3_ref, o_ref):
    x = x_ref[...]
    projT = _dotT(b_ref, x)
    sn, cn = _sincos(projT)
    w0 = w0_ref[...]
    pre = (lax.dot_general(w0[:NF], sn, (((0,), (0,)), ((), ())),
                           preferred_element_type=_f32)
           + lax.dot_general(w0[NF:], cn, (((0,), (0,)), ((), ())),
                             preferred_element_type=_f32))
    h = jnp.tanh(pre + b0_ref[...])
    h = jnp.tanh(_dotT(w1_ref, h) + b1_ref[...])
    h = jnp.tanh(_dotT(w2_ref, h) + b2_ref[...])
    o_ref[...] = _dotT(w3_ref, h) + b3_ref[...]


def _mlp(xtT, B, W0, b0, W1, b1, W2, b2, W3, b3):
    grid = (N_PAD // ROWS,)
    full = lambda r, c: pl.BlockSpec((r, c), lambda i: (0, 0))
    return pl.pallas_call(
        _mlp_body,
        grid=grid,
        in_specs=[
            pl.BlockSpec((2, ROWS), lambda i: (0, i)),
            full(2, NF), full(2 * NF, H), full(H, 1), full(H, H), full(H, 1),
            full(H, H), full(H, 1), full(H, 1), full(1, 1),
        ],
        out_specs=pl.BlockSpec((1, ROWS), lambda i: (0, i)),
        out_shape=jax.ShapeDtypeStruct((1, N_PAD), _f32),
    )(xtT, B, W0, b0.reshape(H, 1), W1, b1.reshape(H, 1),
      W2, b2.reshape(H, 1), W3, b3.reshape(1, 1))


def _hop_body(aa_hbm, ab_hbm, ei_hbm, lw_hbm, a0_hbm, a1_hbm,
              u_tab, b_srcA, b_dstA, b_lwA, b_srcB, b_dstB, b_lwB, sh_buf,
              sem_la, sem_lb, sem_sa, sem_sb):
    c = lax.axis_index("c")
    s = lax.axis_index("s")
    wid = c * 16 + s
    base = wid * EPW

    def _loads(off, bs, bd, blw, sem):
        pltpu.async_copy(ei_hbm.at[pl.ds(off, EC)], bs, sem)
        pltpu.async_copy(ei_hbm.at[pl.ds(E + off, EC)], bd, sem)
        pltpu.async_copy(lw_hbm.at[pl.ds(off, EC)], blw, sem)

    def _wait_loads(bs, bd, blw, sem):
        pltpu.make_async_copy(ei_hbm.at[pl.ds(0, EC)], bs, sem).wait()
        pltpu.make_async_copy(ei_hbm.at[pl.ds(0, EC)], bd, sem).wait()
        pltpu.make_async_copy(lw_hbm.at[pl.ds(0, EC)], blw, sem).wait()

    def _compute(bs, blw):
        @plsc.parallel_loop(0, EC // 16, 1, unroll=10)
        def _(j):
            sl = pl.ds(j * 16, 16)
            blw[sl] = jnp.exp(blw[sl]) * plsc.load_gather(u_tab, [bs[sl]])

    def _wait_scatter(blw, bd, sem):
        pltpu.make_async_copy(blw, sh_buf.at[bd], sem).wait()

    for j in range(SLICE // CSLICE):
        off = s * SLICE + j * CSLICE
        csl = pl.ds(0, CSLICE)
        pltpu.sync_copy(aa_hbm.at[pl.ds(off, CSLICE)], b_lwA.at[csl])
        pltpu.sync_copy(ab_hbm.at[pl.ds(off, CSLICE)], b_lwB.at[csl])

        def _add(i, _):
            sl = pl.ds(i * 16, 16)
            b_lwA[sl] = b_lwA[sl] + b_lwB[sl]
            return 0
        lax.fori_loop(0, CSLICE // 16, _add, 0)

        pltpu.sync_copy(b_lwA.at[csl], sh_buf.at[pl.ds(off, CSLICE)])

    _loads(base, b_srcA, b_dstA, b_lwA, sem_la)

    plsc.subcore_barrier()

    pltpu.sync_copy(sh_buf.at[pl.ds(0, N_TAB)], u_tab)

    plsc.subcore_barrier()

    @pl.when(c == 1)
    def _():
        def _zfill(i, _):
            b_lwB[pl.ds(i * 16, 16)] = jnp.zeros((16,), _f32)
            return 0
        lax.fori_loop(0, CSLICE // 16, _zfill, 0)
        for j in range(SLICE // CSLICE):
            off = s * SLICE + j * CSLICE
            pltpu.sync_copy(b_lwB.at[pl.ds(0, CSLICE)],
                            sh_buf.at[pl.ds(off, CSLICE)])

    plsc.subcore_barrier()

    def _pair(i, _):
        @pl.when(i > 0)
        def _():
            _wait_scatter(b_lwB, b_dstB, sem_sb)
        _loads(base + (2 * i + 1) * EC, b_srcB, b_dstB, b_lwB, sem_lb)
        _wait_loads(b_srcA, b_dstA, b_lwA, sem_la)
        _compute(b_srcA, b_lwA)
        pltpu.async_copy(b_lwA, sh_buf.at[b_dstA], sem_sa, add=True)
        _wait_loads(b_srcB, b_dstB, b_lwB, sem_lb)
        _compute(b_srcB, b_lwB)
        pltpu.async_copy(b_lwB, sh_buf.at[b_dstB], sem_sb, add=True)
        _wait_scatter(b_lwA, b_dstA, sem_sa)

        @pl.when(i < NCH // 2 - 1)
        def _():
            _loads(base + (2 * i + 2) * EC, b_srcA, b_dstA, b_lwA, sem_la)
        return 0
    lax.fori_loop(0, NCH // 2, _pair, 0)
    _wait_scatter(b_lwB, b_dstB, sem_sb)

    plsc.subcore_barrier()

    osl = pl.ds(s * SLICE, SLICE)

    @pl.when(c == 0)
    def _():
        pltpu.sync_copy(sh_buf.at[osl], a0_hbm.at[osl])

    @pl.when(c == 1)
    def _():
        pltpu.sync_copy(sh_buf.at[osl], a1_hbm.at[osl])


_hop = functools.partial(
    pl.kernel,
    out_type=(
        jax.ShapeDtypeStruct((N_PAD,), _f32),
        jax.ShapeDtypeStruct((N_PAD,), _f32),
    ),
    mesh=plsc.VectorSubcoreMesh(core_axis_name="c", subcore_axis_name="s"),
    compiler_params=pltpu.CompilerParams(needs_layout_passes=False),
    scratch_types=[
        pltpu.VMEM((N_TAB,), _f32),
        pltpu.VMEM((EC,), jnp.int32),
        pltpu.VMEM((EC,), jnp.int32),
        pltpu.VMEM((EC,), _f32),
        pltpu.VMEM((EC,), jnp.int32),
        pltpu.VMEM((EC,), jnp.int32),
        pltpu.VMEM((EC,), _f32),
        pltpu.VMEM_SHARED((N_PAD,), _f32),
        pltpu.SemaphoreType.DMA,
        pltpu.SemaphoreType.DMA,
        pltpu.SemaphoreType.DMA,
        pltpu.SemaphoreType.DMA,
    ],
)(_hop_body)


def _fin_body(aa_hbm, ab_hbm, out_hbm, b_a, b_b):
    c = lax.axis_index("c")
    s = lax.axis_index("s")
    wid = c * 16 + s
    for j in range(GSLICE // CSLICE):
        off = wid * GSLICE + j * CSLICE
        pltpu.sync_copy(aa_hbm.at[pl.ds(off, CSLICE)], b_a)
        pltpu.sync_copy(ab_hbm.at[pl.ds(off, CSLICE)], b_b)

        def _add(i, _):
            sl = pl.ds(i * 16, 16)
            b_a[sl] = b_a[sl] + b_b[sl]
            return 0
        lax.fori_loop(0, CSLICE // 16, _add, 0)
        pltpu.sync_copy(b_a, out_hbm.at[pl.ds(off, CSLICE)])


_fin = functools.partial(
    pl.kernel,
    out_type=jax.ShapeDtypeStruct((N_PAD,), _f32),
    mesh=plsc.VectorSubcoreMesh(core_axis_name="c", subcore_axis_name="s"),
    scratch_types=[
        pltpu.VMEM((CSLICE,), _f32),
        pltpu.VMEM((CSLICE,), _f32),
    ],
)(_fin_body)


def kernel(xt, edge_index, log_w, B, W0, b0, W1, b1, W2, b2, W3, b3):
    u = _mlp(xt.T, B, W0, b0, W1, b1, W2, b2, W3, b3).reshape(N_PAD)
    ei = edge_index.reshape(2 * E)
    a0, a1 = u, jnp.zeros((N_PAD,), _f32)
    for _ in range(K):
        a0, a1 = _hop(a0, a1, ei, log_w)
    u = _fin(a0, a1)
    return u[:N].reshape(N, 1)


# speedup vs baseline: 513.9781x; 1.0144x over previous
import functools

import jax
import jax.numpy as jnp
from jax import lax
from jax.experimental import pallas as pl
from jax.experimental.pallas import tpu as pltpu
from jax.experimental.pallas import tpu_sc as plsc

N = 100000
E = 6400000
NF = 64
H = 128
K = 3

N_PAD = 100352
ROWS = 2048
SLICE = N_PAD // 16
CSLICE = SLICE // 4
GSLICE = N_PAD // 32
EPW = E // 32
EC = 4000
NCH = EPW // EC
N_TAB = N

_f32 = jnp.float32


_INV2PI = 0.15915493667125702
_TWOPI = 6.2831854820251465
_SC = (0.9999999378055222, -0.16666621099119006, 0.008332791415928265,
       -0.00019817628169256163, 2.7088274845432636e-06, -2.0697969171923347e-08)
_CC = (0.9999992105908722, -0.499994212222991, 0.04165977670694633,
       -0.001385878635320515, 2.4202894902042305e-05, -2.1972754565835473e-07)


def _sincos(x):
    q = jnp.floor(x * _INV2PI + 0.5)
    r = x - q * _TWOPI
    t = r * r
    s = _SC[0] + t * (_SC[1] + t * (_SC[2] + t * (_SC[3] + t * (_SC[4] + t * _SC[5]))))
    c = _CC[0] + t * (_CC[1] + t * (_CC[2] + t * (_CC[3] + t * (_CC[4] + t * _CC[5]))))
    return s * r, c


def _dotT(w_ref, x):
    return lax.dot_general(w_ref[...], x, (((0,), (0,)), ((), ())),
                           preferred_element_type=_f32)


def _mlp_body(x_ref, b_ref, w0_ref, b0_ref, w1_ref, b1_ref, w2_ref,
              b2_ref, w3_ref, b3_ref, o_ref):
    x = x_ref[...]
    projT = _dotT(b_ref, x)
    sn, cn = _sincos(projT)
    w0 = w0_ref[...]
    pre = (lax.dot_general(w0[:NF], sn, (((0,), (0,)), ((), ())),
                           preferred_element_type=_f32)
           + lax.dot_general(w0[NF:], cn, (((0,), (0,)), ((), ())),
                             preferred_element_type=_f32))
    h = jnp.tanh(pre + b0_ref[...])
    h = jnp.tanh(_dotT(w1_ref, h) + b1_ref[...])
    h = jnp.tanh(_dotT(w2_ref, h) + b2_ref[...])
    o_ref[...] = _dotT(w3_ref, h) + b3_ref[...]


def _mlp(xtT, B, W0, b0, W1, b1, W2, b2, W3, b3):
    grid = (N_PAD // ROWS,)
    full = lambda r, c: pl.BlockSpec((r, c), lambda i: (0, 0))
    return pl.pallas_call(
        _mlp_body,
        grid=grid,
        in_specs=[
            pl.BlockSpec((2, ROWS), lambda i: (0, i)),
            full(2, NF), full(2 * NF, H), full(H, 1), full(H, H), full(H, 1),
            full(H, H), full(H, 1), full(H, 1), full(1, 1),
        ],
        out_specs=pl.BlockSpec((1, ROWS), lambda i: (0, i)),
        out_shape=jax.ShapeDtypeStruct((1, N_PAD), _f32),
    )(xtT, B, W0, b0.reshape(H, 1), W1, b1.reshape(H, 1),
      W2, b2.reshape(H, 1), W3, b3.reshape(1, 1))


def _hop_body(aa_hbm, ab_hbm, ei_hbm, lw_hbm, a0_hbm, a1_hbm,
              u_tab, b_srcA, b_dstA, b_lwA, b_srcB, b_dstB, b_lwB, sh_buf,
              sem_la, sem_lb, sem_sa, sem_sb):
    c = lax.axis_index("c")
    s = lax.axis_index("s")
    wid = c * 16 + s
    base = wid * EPW

    def _loads(off, bs, bd, blw, sem):
        pltpu.async_copy(ei_hbm.at[pl.ds(off, EC)], bs, sem)
        pltpu.async_copy(ei_hbm.at[pl.ds(E + off, EC)], bd, sem)
        pltpu.async_copy(lw_hbm.at[pl.ds(off, EC)], blw, sem)

    def _wait_loads(bs, bd, blw, sem):
        pltpu.make_async_copy(ei_hbm.at[pl.ds(0, EC)], bs, sem).wait()
        pltpu.make_async_copy(ei_hbm.at[pl.ds(0, EC)], bd, sem).wait()
        pltpu.make_async_copy(lw_hbm.at[pl.ds(0, EC)], blw, sem).wait()

    def _compute(bs, blw):
        @plsc.parallel_loop(0, EC // 16, 1, unroll=25)
        def _(j):
            sl = pl.ds(j * 16, 16)
            blw[sl] = jnp.exp(blw[sl]) * plsc.load_gather(u_tab, [bs[sl]])

    def _wait_scatter(blw, bd, sem):
        pltpu.make_async_copy(blw, sh_buf.at[bd], sem).wait()

    for j in range(SLICE // CSLICE):
        off = s * SLICE + j * CSLICE
        csl = pl.ds(0, CSLICE)
        pltpu.async_copy(aa_hbm.at[pl.ds(off, CSLICE)], b_lwA.at[csl], sem_la)
        pltpu.async_copy(ab_hbm.at[pl.ds(off, CSLICE)], b_lwB.at[csl], sem_lb)
        pltpu.make_async_copy(aa_hbm.at[pl.ds(off, CSLICE)], b_lwA.at[csl],
                              sem_la).wait()
        pltpu.make_async_copy(ab_hbm.at[pl.ds(off, CSLICE)], b_lwB.at[csl],
                              sem_lb).wait()

        def _add(i, _):
            sl = pl.ds(i * 16, 16)
            b_lwA[sl] = b_lwA[sl] + b_lwB[sl]
            return 0
        lax.fori_loop(0, CSLICE // 16, _add, 0)

        pltpu.sync_copy(b_lwA.at[csl], sh_buf.at[pl.ds(off, CSLICE)])

    _loads(base, b_srcA, b_dstA, b_lwA, sem_la)

    plsc.subcore_barrier()

    pltpu.sync_copy(sh_buf.at[pl.ds(0, N_TAB)], u_tab)

    plsc.subcore_barrier()

    @pl.when(c == 1)
    def _():
        def _zfill(i, _):
            b_lwB[pl.ds(i * 16, 16)] = jnp.zeros((16,), _f32)
            return 0
        lax.fori_loop(0, CSLICE // 16, _zfill, 0)
        for j in range(SLICE // CSLICE):
            off = s * SLICE + j * CSLICE
            pltpu.sync_copy(b_lwB.at[pl.ds(0, CSLICE)],
                            sh_buf.at[pl.ds(off, CSLICE)])

    _loads(base + EC, b_srcB, b_dstB, b_lwB, sem_lb)

    plsc.subcore_barrier()

    def _pair(i, _):
        @pl.when(i > 0)
        def _():
            _wait_scatter(b_lwB, b_dstB, sem_sb)
            _loads(base + (2 * i + 1) * EC, b_srcB, b_dstB, b_lwB, sem_lb)
        _wait_loads(b_srcA, b_dstA, b_lwA, sem_la)
        _compute(b_srcA, b_lwA)
        pltpu.async_copy(b_lwA, sh_buf.at[b_dstA], sem_sa, add=True)
        _wait_loads(b_srcB, b_dstB, b_lwB, sem_lb)
        _compute(b_srcB, b_lwB)
        pltpu.async_copy(b_lwB, sh_buf.at[b_dstB], sem_sb, add=True)
        _wait_scatter(b_lwA, b_dstA, sem_sa)

        @pl.when(i < NCH // 2 - 1)
        def _():
            _loads(base + (2 * i + 2) * EC, b_srcA, b_dstA, b_lwA, sem_la)
        return 0
    lax.fori_loop(0, NCH // 2, _pair, 0)
    _wait_scatter(b_lwB, b_dstB, sem_sb)

    plsc.subcore_barrier()

    osl = pl.ds(s * SLICE, SLICE)

    @pl.when(c == 0)
    def _():
        pltpu.sync_copy(sh_buf.at[osl], a0_hbm.at[osl])

    @pl.when(c == 1)
    def _():
        pltpu.sync_copy(sh_buf.at[osl], a1_hbm.at[osl])


_hop = functools.partial(
    pl.kernel,
    out_type=(
        jax.ShapeDtypeStruct((N_PAD,), _f32),
        jax.ShapeDtypeStruct((N_PAD,), _f32),
    ),
    mesh=plsc.VectorSubcoreMesh(core_axis_name="c", subcore_axis_name="s"),
    compiler_params=pltpu.CompilerParams(needs_layout_passes=False),
    scratch_types=[
        pltpu.VMEM((N_TAB,), _f32),
        pltpu.VMEM((EC,), jnp.int32),
        pltpu.VMEM((EC,), jnp.int32),
        pltpu.VMEM((EC,), _f32),
        pltpu.VMEM((EC,), jnp.int32),
        pltpu.VMEM((EC,), jnp.int32),
        pltpu.VMEM((EC,), _f32),
        pltpu.VMEM_SHARED((N_PAD,), _f32),
        pltpu.SemaphoreType.DMA,
        pltpu.SemaphoreType.DMA,
        pltpu.SemaphoreType.DMA,
        pltpu.SemaphoreType.DMA,
    ],
)(_hop_body)


def _fin_body(aa_hbm, ab_hbm, out_hbm, b_a, b_b):
    c = lax.axis_index("c")
    s = lax.axis_index("s")
    wid = c * 16 + s
    for j in range(GSLICE // CSLICE):
        off = wid * GSLICE + j * CSLICE
        pltpu.sync_copy(aa_hbm.at[pl.ds(off, CSLICE)], b_a)
        pltpu.sync_copy(ab_hbm.at[pl.ds(off, CSLICE)], b_b)

        def _add(i, _):
            sl = pl.ds(i * 16, 16)
            b_a[sl] = b_a[sl] + b_b[sl]
            return 0
        lax.fori_loop(0, CSLICE // 16, _add, 0)
        pltpu.sync_copy(b_a, out_hbm.at[pl.ds(off, CSLICE)])


_fin = functools.partial(
    pl.kernel,
    out_type=jax.ShapeDtypeStruct((N_PAD,), _f32),
    mesh=plsc.VectorSubcoreMesh(core_axis_name="c", subcore_axis_name="s"),
    scratch_types=[
        pltpu.VMEM((CSLICE,), _f32),
        pltpu.VMEM((CSLICE,), _f32),
    ],
)(_fin_body)


def kernel(xt, edge_index, log_w, B, W0, b0, W1, b1, W2, b2, W3, b3):
    u = _mlp(xt.T, B, W0, b0, W1, b1, W2, b2, W3, b3).reshape(N_PAD)
    ei = edge_index.reshape(2 * E)
    a0, a1 = u, jnp.zeros((N_PAD,), _f32)
    for _ in range(K):
        a0, a1 = _hop(a0, a1, ei, log_w)
    u = _fin(a0, a1)
    return u[:N].reshape(N, 1)


# speedup vs baseline: 518.9275x; 1.0096x over previous
import functools

import jax
import jax.numpy as jnp
from jax import lax
from jax.experimental import pallas as pl
from jax.experimental.pallas import tpu as pltpu
from jax.experimental.pallas import tpu_sc as plsc

N = 100000
E = 6400000
NF = 64
H = 128
K = 3

N_PAD = 100352
ROWS = 2048
SLICE = N_PAD // 16
CSLICE = SLICE // 4
GSLICE = N_PAD // 32
EPW = E // 32
EC = 4000
NCH = EPW // EC
N_TAB = N

_f32 = jnp.float32


_INV2PI = 0.15915493667125702
_TWOPI = 6.2831854820251465
_SC = (0.9999999378055222, -0.16666621099119006, 0.008332791415928265,
       -0.00019817628169256163, 2.7088274845432636e-06, -2.0697969171923347e-08)
_CC = (0.9999992105908722, -0.499994212222991, 0.04165977670694633,
       -0.001385878635320515, 2.4202894902042305e-05, -2.1972754565835473e-07)


def _sincos(x):
    q = jnp.floor(x * _INV2PI + 0.5)
    r = x - q * _TWOPI
    t = r * r
    s = _SC[0] + t * (_SC[1] + t * (_SC[2] + t * (_SC[3] + t * (_SC[4] + t * _SC[5]))))
    c = _CC[0] + t * (_CC[1] + t * (_CC[2] + t * (_CC[3] + t * (_CC[4] + t * _CC[5]))))
    return s * r, c


def _dotT(w_ref, x):
    return lax.dot_general(w_ref[...], x, (((0,), (0,)), ((), ())),
                           preferred_element_type=_f32)


def _mlp_body(x_ref, b_ref, w0_ref, b0_ref, w1_ref, b1_ref, w2_ref,
              b2_ref, w3_ref, b3_ref, o_ref):
    x = x_ref[...]
    projT = _dotT(b_ref, x)
    sn, cn = _sincos(projT)
    w0 = w0_ref[...]
    pre = (lax.dot_general(w0[:NF], sn, (((0,), (0,)), ((), ())),
                           preferred_element_type=_f32)
           + lax.dot_general(w0[NF:], cn, (((0,), (0,)), ((), ())),
                             preferred_element_type=_f32))
    h = jnp.tanh(pre + b0_ref[...])
    h = jnp.tanh(_dotT(w1_ref, h) + b1_ref[...])
    h = jnp.tanh(_dotT(w2_ref, h) + b2_ref[...])
    o_ref[...] = _dotT(w3_ref, h) + b3_ref[...]


def _mlp(xtT, B, W0, b0, W1, b1, W2, b2, W3, b3):
    grid = (N_PAD // ROWS,)
    full = lambda r, c: pl.BlockSpec((r, c), lambda i: (0, 0))
    return pl.pallas_call(
        _mlp_body,
        grid=grid,
        in_specs=[
            pl.BlockSpec((2, ROWS), lambda i: (0, i)),
            full(2, NF), full(2 * NF, H), full(H, 1), full(H, H), full(H, 1),
            full(H, H), full(H, 1), full(H, 1), full(1, 1),
        ],
        out_specs=pl.BlockSpec((1, ROWS), lambda i: (0, i)),
        out_shape=jax.ShapeDtypeStruct((1, N_PAD), _f32),
    )(xtT, B, W0, b0.reshape(H, 1), W1, b1.reshape(H, 1),
      W2, b2.reshape(H, 1), W3, b3.reshape(1, 1))


def _hop_body(aa_hbm, ab_hbm, ei_hbm, lw_hbm, a0_hbm, a1_hbm,
              u_tab, b_srcA, b_dstA, b_lwA, b_srcB, b_dstB, b_lwB, sh_buf,
              sem_la, sem_lb, sem_sa, sem_sb, first=False):
    c = lax.axis_index("c")
    s = lax.axis_index("s")
    wid = c * 16 + s
    base = wid * EPW

    def _loads(off, bs, bd, blw, sem):
        pltpu.async_copy(ei_hbm.at[pl.ds(off, EC)], bs, sem)
        pltpu.async_copy(ei_hbm.at[pl.ds(E + off, EC)], bd, sem)
        pltpu.async_copy(lw_hbm.at[pl.ds(off, EC)], blw, sem)

    def _wait_loads(bs, bd, blw, sem):
        pltpu.make_async_copy(ei_hbm.at[pl.ds(0, EC)], bs, sem).wait()
        pltpu.make_async_copy(ei_hbm.at[pl.ds(0, EC)], bd, sem).wait()
        pltpu.make_async_copy(lw_hbm.at[pl.ds(0, EC)], blw, sem).wait()

    def _compute(bs, blw):
        @plsc.parallel_loop(0, EC // 16, 1, unroll=25)
        def _(j):
            sl = pl.ds(j * 16, 16)
            blw[sl] = jnp.exp(blw[sl]) * plsc.load_gather(u_tab, [bs[sl]])

    def _wait_scatter(blw, bd, sem):
        pltpu.make_async_copy(blw, sh_buf.at[bd], sem).wait()

    if first:
        osl0 = pl.ds(s * SLICE, SLICE)
        pltpu.sync_copy(aa_hbm.at[osl0], sh_buf.at[osl0])
    else:
        for j in range(SLICE // CSLICE):
            off = s * SLICE + j * CSLICE
            csl = pl.ds(0, CSLICE)
            pltpu.async_copy(aa_hbm.at[pl.ds(off, CSLICE)], b_lwA.at[csl],
                             sem_la)
            pltpu.async_copy(ab_hbm.at[pl.ds(off, CSLICE)], b_lwB.at[csl],
                             sem_lb)
            pltpu.make_async_copy(aa_hbm.at[pl.ds(off, CSLICE)],
                                  b_lwA.at[csl], sem_la).wait()
            pltpu.make_async_copy(ab_hbm.at[pl.ds(off, CSLICE)],
                                  b_lwB.at[csl], sem_lb).wait()

            def _add(i, _):
                sl = pl.ds(i * 16, 16)
                b_lwA[sl] = b_lwA[sl] + b_lwB[sl]
                return 0
            lax.fori_loop(0, CSLICE // 16, _add, 0)

            pltpu.sync_copy(b_lwA.at[csl], sh_buf.at[pl.ds(off, CSLICE)])

    _loads(base, b_srcA, b_dstA, b_lwA, sem_la)

    plsc.subcore_barrier()

    pltpu.sync_copy(sh_buf.at[pl.ds(0, N_TAB)], u_tab)

    plsc.subcore_barrier()

    @pl.when(c == 1)
    def _():
        def _zfill(i, _):
            b_lwB[pl.ds(i * 16, 16)] = jnp.zeros((16,), _f32)
            return 0
        lax.fori_loop(0, CSLICE // 16, _zfill, 0)
        for j in range(SLICE // CSLICE):
            off = s * SLICE + j * CSLICE
            pltpu.sync_copy(b_lwB.at[pl.ds(0, CSLICE)],
                            sh_buf.at[pl.ds(off, CSLICE)])

    _loads(base + EC, b_srcB, b_dstB, b_lwB, sem_lb)

    plsc.subcore_barrier()

    def _pair(i, _):
        @pl.when(i > 0)
        def _():
            _wait_scatter(b_lwB, b_dstB, sem_sb)
            _loads(base + (2 * i + 1) * EC, b_srcB, b_dstB, b_lwB, sem_lb)
        _wait_loads(b_srcA, b_dstA, b_lwA, sem_la)
        _compute(b_srcA, b_lwA)
        pltpu.async_copy(b_lwA, sh_buf.at[b_dstA], sem_sa, add=True)
        _wait_loads(b_srcB, b_dstB, b_lwB, sem_lb)
        _compute(b_srcB, b_lwB)
        pltpu.async_copy(b_lwB, sh_buf.at[b_dstB], sem_sb, add=True)
        _wait_scatter(b_lwA, b_dstA, sem_sa)

        @pl.when(i < NCH // 2 - 1)
        def _():
            _loads(base + (2 * i + 2) * EC, b_srcA, b_dstA, b_lwA, sem_la)
        return 0
    lax.fori_loop(0, NCH // 2, _pair, 0)
    _wait_scatter(b_lwB, b_dstB, sem_sb)

    plsc.subcore_barrier()

    osl = pl.ds(s * SLICE, SLICE)

    @pl.when(c == 0)
    def _():
        pltpu.sync_copy(sh_buf.at[osl], a0_hbm.at[osl])

    @pl.when(c == 1)
    def _():
        pltpu.sync_copy(sh_buf.at[osl], a1_hbm.at[osl])


def _hop0_body(aa_hbm, ei_hbm, lw_hbm, a0_hbm, a1_hbm,
               u_tab, b_srcA, b_dstA, b_lwA, b_srcB, b_dstB, b_lwB, sh_buf,
               sem_la, sem_lb, sem_sa, sem_sb):
    _hop_body(aa_hbm, aa_hbm, ei_hbm, lw_hbm, a0_hbm, a1_hbm,
              u_tab, b_srcA, b_dstA, b_lwA, b_srcB, b_dstB, b_lwB, sh_buf,
              sem_la, sem_lb, sem_sa, sem_sb, first=True)


_HOP_SCRATCH = [
    pltpu.VMEM((N_TAB,), _f32),
    pltpu.VMEM((EC,), jnp.int32),
    pltpu.VMEM((EC,), jnp.int32),
    pltpu.VMEM((EC,), _f32),
    pltpu.VMEM((EC,), jnp.int32),
    pltpu.VMEM((EC,), jnp.int32),
    pltpu.VMEM((EC,), _f32),
    pltpu.VMEM_SHARED((N_PAD,), _f32),
    pltpu.SemaphoreType.DMA,
    pltpu.SemaphoreType.DMA,
    pltpu.SemaphoreType.DMA,
    pltpu.SemaphoreType.DMA,
]

_hop0 = functools.partial(
    pl.kernel,
    out_type=(
        jax.ShapeDtypeStruct((N_PAD,), _f32),
        jax.ShapeDtypeStruct((N_PAD,), _f32),
    ),
    mesh=plsc.VectorSubcoreMesh(core_axis_name="c", subcore_axis_name="s"),
    compiler_params=pltpu.CompilerParams(needs_layout_passes=False),
    scratch_types=_HOP_SCRATCH,
)(_hop0_body)

_hop = functools.partial(
    pl.kernel,
    out_type=(
        jax.ShapeDtypeStruct((N_PAD,), _f32),
        jax.ShapeDtypeStruct((N_PAD,), _f32),
    ),
    mesh=plsc.VectorSubcoreMesh(core_axis_name="c", subcore_axis_name="s"),
    compiler_params=pltpu.CompilerParams(needs_layout_passes=False),
    scratch_types=_HOP_SCRATCH,
)(_hop_body)


def _fin_body(aa_hbm, ab_hbm, out_hbm, b_a, b_b):
    c = lax.axis_index("c")
    s = lax.axis_index("s")
    wid = c * 16 + s
    for j in range(GSLICE // CSLICE):
        off = wid * GSLICE + j * CSLICE
        pltpu.sync_copy(aa_hbm.at[pl.ds(off, CSLICE)], b_a)
        pltpu.sync_copy(ab_hbm.at[pl.ds(off, CSLICE)], b_b)

        def _add(i, _):
            sl = pl.ds(i * 16, 16)
            b_a[sl] = b_a[sl] + b_b[sl]
            return 0
        lax.fori_loop(0, CSLICE // 16, _add, 0)
        pltpu.sync_copy(b_a, out_hbm.at[pl.ds(off, CSLICE)])


_fin = functools.partial(
    pl.kernel,
    out_type=jax.ShapeDtypeStruct((N_PAD,), _f32),
    mesh=plsc.VectorSubcoreMesh(core_axis_name="c", subcore_axis_name="s"),
    scratch_types=[
        pltpu.VMEM((CSLICE,), _f32),
        pltpu.VMEM((CSLICE,), _f32),
    ],
)(_fin_body)


def kernel(xt, edge_index, log_w, B, W0, b0, W1, b1, W2, b2, W3, b3):
    u = _mlp(xt.T, B, W0, b0, W1, b1, W2, b2, W3, b3).reshape(N_PAD)
    ei = edge_index.reshape(2 * E)
    a0, a1 = _hop0(u, ei, log_w)
    for _ in range(K - 1):
        a0, a1 = _hop(a0, a1, ei, log_w)
    u = _fin(a0, a1)
    return u[:N].reshape(N, 1)
